# baseline (speedup 1.0000x reference)
import functools

import jax
import jax.numpy as jnp
from jax import lax
from jax.experimental import pallas as pl
from jax.experimental.pallas import tpu as pltpu

E = 800000
N = 50000
BE = 4000
BN = 2000

_PREC = lax.Precision.HIGHEST


def _dot(a, b):
    return lax.dot_general(a, b, (((1,), (0,)), ((), ())),
                           precision=_PREC, preferred_element_type=jnp.float32)


def _edge_block_kernel(xi_ref, xj_ref, ii_ref, ij_ref, ea_ref, aea_ref,
                       w_refs, ue_ref, fut_ref, past_ref):
    (eu_w1, eu_b1, eu_w2, eu_b2, eu_w3, eu_b3,
     f_w1, f_b1, f_w2, f_b2,
     p_w1, p_b1, p_w2, p_b2) = w_refs
    x_i = xi_ref[...]
    x_j = xj_ref[...]
    init_i = ii_ref[...]
    init_j = ij_ref[...]
    ea = ea_ref[...]
    aea = aea_ref[...]

    feats = jnp.concatenate([x_i, x_j, ea, aea], axis=1)
    h = jax.nn.relu(_dot(feats, eu_w1[...]) + eu_b1[...])
    h = jax.nn.relu(_dot(h, eu_w2[...]) + eu_b2[...])
    ue = _dot(h, eu_w3[...]) + eu_b3[...]
    ue_ref[...] = ue

    ffeat = jnp.concatenate([x_i, ue, init_i], axis=1)
    hf = jax.nn.relu(_dot(ffeat, f_w1[...]) + f_b1[...])
    fut_ref[...] = _dot(hf, f_w2[...]) + f_b2[...]

    pfeat = jnp.concatenate([x_j, ue, init_j], axis=1)
    hp = jax.nn.relu(_dot(pfeat, p_w1[...]) + p_b1[...])
    past_ref[...] = _dot(hp, p_w2[...]) + p_b2[...]


def _node_block_kernel(msg_ref, w_refs, out_ref):
    (w1, b1, w2, b2, w3, b3) = w_refs
    m = msg_ref[...]
    h = jax.nn.relu(_dot(m, w1[...]) + b1[...])
    h = jax.nn.relu(_dot(h, w2[...]) + b2[...])
    out_ref[...] = _dot(h, w3[...]) + b3[...]


def _edge_stage(x_i, x_j, init_i, init_j, edge_attr, att_edge_attr, wflat):
    nblk = E // BE
    eb = lambda i: (i, 0)
    wspec = [pl.BlockSpec(w.shape, lambda i, nd=w.ndim: (0,) * nd) for w in wflat]
    grid_spec = pltpu.PrefetchScalarGridSpec(
        num_scalar_prefetch=0,
        grid=(nblk,),
        in_specs=[
            pl.BlockSpec((BE, 96), eb),
            pl.BlockSpec((BE, 96), eb),
            pl.BlockSpec((BE, 96), eb),
            pl.BlockSpec((BE, 96), eb),
            pl.BlockSpec((BE, 64), eb),
            pl.BlockSpec((BE, 64), eb),
            wspec,
        ],
        out_specs=[
            pl.BlockSpec((BE, 64), eb),
            pl.BlockSpec((BE, 128), eb),
            pl.BlockSpec((BE, 128), eb),
        ],
    )
    return pl.pallas_call(
        _edge_block_kernel,
        grid_spec=grid_spec,
        out_shape=[
            jax.ShapeDtypeStruct((E, 64), jnp.float32),
            jax.ShapeDtypeStruct((E, 128), jnp.float32),
            jax.ShapeDtypeStruct((E, 128), jnp.float32),
        ],
    )(x_i, x_j, init_i, init_j, edge_attr, att_edge_attr, wflat)


def _node_stage(messages, wflat):
    nblk = N // BN
    wspec = [pl.BlockSpec(w.shape, lambda i, nd=w.ndim: (0,) * nd) for w in wflat]
    grid_spec = pltpu.PrefetchScalarGridSpec(
        num_scalar_prefetch=0,
        grid=(nblk,),
        in_specs=[pl.BlockSpec((BN, 256), lambda i: (i, 0)), wspec],
        out_specs=pl.BlockSpec((BN, 96), lambda i: (i, 0)),
    )
    return pl.pallas_call(
        _node_block_kernel,
        grid_spec=grid_spec,
        out_shape=jax.ShapeDtypeStruct((N, 96), jnp.float32),
    )(messages, wflat)


def kernel(x, edge_attr, initial_x, att_edge_attr, params, edge_index):
    rows = edge_index[0]
    cols = edge_index[1]
    x_j = jnp.take(x, rows, axis=0)
    x_i = jnp.take(x, cols, axis=0)
    init_j = jnp.take(initial_x, rows, axis=0)
    init_i = jnp.take(initial_x, cols, axis=0)

    eu = params["edge_update"]
    fm = params["create_future_msgs"]
    pm = params["create_past_msgs"]
    cb = params["combine_future_past"]

    edge_w = (eu[0][0], eu[0][1], eu[1][0], eu[1][1], eu[2][0], eu[2][1],
              fm[0][0], fm[0][1], fm[1][0], fm[1][1],
              pm[0][0], pm[0][1], pm[1][0], pm[1][1])
    ue, fut, past = _edge_stage(x_i, x_j, init_i, init_j,
                                edge_attr, att_edge_attr, list(edge_w))

    messages_past = jax.ops.segment_sum(past, cols, num_segments=N)
    messages_future = jax.ops.segment_sum(fut, rows, num_segments=N)
    messages = jnp.concatenate([messages_past, messages_future], axis=1)

    node_w = [cb[0][0], cb[0][1], cb[1][0], cb[1][1], cb[2][0], cb[2][1]]
    updated_nodes = _node_stage(messages, node_w)
    return (updated_nodes, ue)


# speedup vs baseline: 1.6429x; 1.6429x over previous
import functools

import jax
import jax.numpy as jnp
from jax import lax
from jax.experimental import pallas as pl
from jax.experimental.pallas import tpu as pltpu

E = 800000
N = 50000
BE = 4000
BN = 2000

_PREC = lax.Precision.DEFAULT


def _dot(a, b):
    return lax.dot_general(a, b, (((1,), (0,)), ((), ())),
                           precision=_PREC, preferred_element_type=jnp.float32)


def _edge_block_kernel(xi_ref, xj_ref, ii_ref, ij_ref, ea_ref, aea_ref,
                       w_refs, ue_ref, fut_ref, past_ref):
    (eu_w1, eu_b1, eu_w2, eu_b2, eu_w3, eu_b3,
     f_w1, f_b1, f_w2, f_b2,
     p_w1, p_b1, p_w2, p_b2) = w_refs
    x_i = xi_ref[...]
    x_j = xj_ref[...]
    init_i = ii_ref[...]
    init_j = ij_ref[...]
    ea = ea_ref[...]
    aea = aea_ref[...]

    feats = jnp.concatenate([x_i, x_j, ea, aea], axis=1)
    h = jax.nn.relu(_dot(feats, eu_w1[...]) + eu_b1[...])
    h = jax.nn.relu(_dot(h, eu_w2[...]) + eu_b2[...])
    ue = _dot(h, eu_w3[...]) + eu_b3[...]
    ue_ref[...] = ue

    ffeat = jnp.concatenate([x_i, ue, init_i], axis=1)
    hf = jax.nn.relu(_dot(ffeat, f_w1[...]) + f_b1[...])
    fut_ref[...] = _dot(hf, f_w2[...]) + f_b2[...]

    pfeat = jnp.concatenate([x_j, ue, init_j], axis=1)
    hp = jax.nn.relu(_dot(pfeat, p_w1[...]) + p_b1[...])
    past_ref[...] = _dot(hp, p_w2[...]) + p_b2[...]


def _node_block_kernel(msg_ref, w_refs, out_ref):
    (w1, b1, w2, b2, w3, b3) = w_refs
    m = msg_ref[...]
    h = jax.nn.relu(_dot(m, w1[...]) + b1[...])
    h = jax.nn.relu(_dot(h, w2[...]) + b2[...])
    out_ref[...] = _dot(h, w3[...]) + b3[...]


def _edge_stage(x_i, x_j, init_i, init_j, edge_attr, att_edge_attr, wflat):
    nblk = E // BE
    eb = lambda i: (i, 0)
    wspec = [pl.BlockSpec(w.shape, lambda i, nd=w.ndim: (0,) * nd) for w in wflat]
    grid_spec = pltpu.PrefetchScalarGridSpec(
        num_scalar_prefetch=0,
        grid=(nblk,),
        in_specs=[
            pl.BlockSpec((BE, 96), eb),
            pl.BlockSpec((BE, 96), eb),
            pl.BlockSpec((BE, 96), eb),
            pl.BlockSpec((BE, 96), eb),
            pl.BlockSpec((BE, 64), eb),
            pl.BlockSpec((BE, 64), eb),
            wspec,
        ],
        out_specs=[
            pl.BlockSpec((BE, 64), eb),
            pl.BlockSpec((BE, 128), eb),
            pl.BlockSpec((BE, 128), eb),
        ],
    )
    return pl.pallas_call(
        _edge_block_kernel,
        grid_spec=grid_spec,
        out_shape=[
            jax.ShapeDtypeStruct((E, 64), jnp.float32),
            jax.ShapeDtypeStruct((E, 128), jnp.float32),
            jax.ShapeDtypeStruct((E, 128), jnp.float32),
        ],
    )(x_i, x_j, init_i, init_j, edge_attr, att_edge_attr, wflat)


def _node_stage(messages, wflat):
    nblk = N // BN
    wspec = [pl.BlockSpec(w.shape, lambda i, nd=w.ndim: (0,) * nd) for w in wflat]
    grid_spec = pltpu.PrefetchScalarGridSpec(
        num_scalar_prefetch=0,
        grid=(nblk,),
        in_specs=[pl.BlockSpec((BN, 256), lambda i: (i, 0)), wspec],
        out_specs=pl.BlockSpec((BN, 96), lambda i: (i, 0)),
    )
    return pl.pallas_call(
        _node_block_kernel,
        grid_spec=grid_spec,
        out_shape=jax.ShapeDtypeStruct((N, 96), jnp.float32),
    )(messages, wflat)


def kernel(x, edge_attr, initial_x, att_edge_attr, params, edge_index):
    rows = edge_index[0]
    cols = edge_index[1]
    x_j = jnp.take(x, rows, axis=0)
    x_i = jnp.take(x, cols, axis=0)
    init_j = jnp.take(initial_x, rows, axis=0)
    init_i = jnp.take(initial_x, cols, axis=0)

    eu = params["edge_update"]
    fm = params["create_future_msgs"]
    pm = params["create_past_msgs"]
    cb = params["combine_future_past"]

    edge_w = (eu[0][0], eu[0][1], eu[1][0], eu[1][1], eu[2][0], eu[2][1],
              fm[0][0], fm[0][1], fm[1][0], fm[1][1],
              pm[0][0], pm[0][1], pm[1][0], pm[1][1])
    ue, fut, past = _edge_stage(x_i, x_j, init_i, init_j,
                                edge_attr, att_edge_attr, list(edge_w))

    messages_past = jax.ops.segment_sum(past, cols, num_segments=N)
    messages_future = jax.ops.segment_sum(fut, rows, num_segments=N)
    messages = jnp.concatenate([messages_past, messages_future], axis=1)

    node_w = [cb[0][0], cb[0][1], cb[1][0], cb[1][1], cb[2][0], cb[2][1]]
    updated_nodes = _node_stage(messages, node_w)
    return (updated_nodes, ue)
